# Initial kernel scaffold; baseline (speedup 1.0000x reference)
import functools

import jax
import jax.numpy as jnp
from jax import lax
from jax.experimental import pallas as pl
from jax.experimental.pallas import tpu as pltpu
from jax.experimental.pallas import tpu_sc as plsc

N = 10000
E = 320000
D = 128

NC = 2
NS = 16
NW = NC * NS
E_PER_W = E // NW
CHUNK = 40
N_CHUNKS = E_PER_W // CHUNK
ROWS_PER_TILE = N // NS
ZCHUNK = 25

_LANES = 16
_VPR = D // _LANES


def _linear_body(x_ref, w_ref, b_ref, o_ref):
    o_ref[...] = (
        jnp.dot(x_ref[...], w_ref[...], preferred_element_type=jnp.float32)
        + b_ref[...]
    )


def _linear(x, W, b):
    bm = 2000
    return pl.pallas_call(
        _linear_body,
        grid=(N // bm,),
        in_specs=[
            pl.BlockSpec((bm, D), lambda i: (i, 0)),
            pl.BlockSpec((D, D), lambda i: (0, 0)),
            pl.BlockSpec((1, D), lambda i: (0, 0)),
        ],
        out_specs=pl.BlockSpec((bm, D), lambda i: (i, 0)),
        out_shape=jax.ShapeDtypeStruct((N, D), jnp.float32),
    )(x, W, b.reshape(1, D))


def _sc_spmm_body(h_hbm, src_hbm, dst_hbm, adj_hbm, out_hbm,
                  src_v, adj_v, dst_v, rows_v, acc_sh, sem):
    c = lax.axis_index("c")
    s = lax.axis_index("s")
    wid = s * NC + c

    zero16 = jnp.zeros((_LANES,), jnp.float32)
    for i in range(ZCHUNK):
        for j in range(_VPR):
            rows_v[i, pl.ds(j * _LANES, _LANES)] = zero16
    row0 = s * ROWS_PER_TILE

    def zero_body(i, _):
        pltpu.sync_copy(
            rows_v.at[pl.ds(0, ZCHUNK)],
            acc_sh.at[pl.ds(row0 + i * ZCHUNK, ZCHUNK)],
        )
        return ()

    lax.fori_loop(0, ROWS_PER_TILE // ZCHUNK, zero_body, ())
    plsc.subcore_barrier()

    e0 = wid * E_PER_W
    pltpu.sync_copy(src_hbm.at[pl.ds(e0, E_PER_W)], src_v)
    pltpu.sync_copy(adj_hbm.at[pl.ds(e0, E_PER_W)], adj_v)

    def chunk_body(t, _):
        base = t * CHUNK
        pltpu.sync_copy(dst_hbm.at[pl.ds(e0 + base, CHUNK)], dst_v)
        pltpu.async_copy(
            h_hbm.at[src_v.at[pl.ds(base, CHUNK)]], rows_v, sem
        ).wait()
        for i in range(CHUNK):
            a = plsc.load_gather(
                adj_v, [jnp.full((_LANES,), base + i, jnp.int32)]
            )
            for j in range(_VPR):
                sl = pl.ds(j * _LANES, _LANES)
                rows_v[i, sl] = rows_v[i, sl] * a
        pltpu.sync_copy(rows_v, acc_sh.at[dst_v], add=True)
        return ()

    lax.fori_loop(0, N_CHUNKS, chunk_body, ())
    plsc.subcore_barrier()

    pltpu.sync_copy(
        acc_sh.at[pl.ds(row0, ROWS_PER_TILE)],
        out_hbm.at[c].at[pl.ds(row0, ROWS_PER_TILE)],
    )


_sc_spmm = functools.partial(
    pl.kernel,
    out_type=jax.ShapeDtypeStruct((NC, N, D), jnp.float32),
    mesh=plsc.VectorSubcoreMesh(core_axis_name="c", subcore_axis_name="s"),
    scratch_types=[
        pltpu.VMEM((E_PER_W,), jnp.int32),
        pltpu.VMEM((E_PER_W,), jnp.float32),
        pltpu.VMEM((CHUNK,), jnp.int32),
        pltpu.VMEM((CHUNK, D), jnp.float32),
        pltpu.VMEM_SHARED((N, D), jnp.float32),
        pltpu.SemaphoreType.DMA,
    ],
)(_sc_spmm_body)


def _elu_body(p_ref, o_ref):
    t = p_ref[0] + p_ref[1]
    o_ref[...] = jnp.where(t > 0, t, jnp.expm1(t))


def _elu_combine(partials):
    bm = 2000
    return pl.pallas_call(
        _elu_body,
        grid=(N // bm,),
        in_specs=[pl.BlockSpec((NC, bm, D), lambda i: (0, i, 0))],
        out_specs=pl.BlockSpec((bm, D), lambda i: (i, 0)),
        out_shape=jax.ShapeDtypeStruct((N, D), jnp.float32),
    )(partials)


def kernel(x, edge_index, adj_values, W, b):
    h = _linear(x, W, b)
    dst = edge_index[0]
    src = edge_index[1]
    partials = _sc_spmm(h, src, dst, adj_values)
    return _elu_combine(partials)


# speedup vs baseline: 5.7798x; 5.7798x over previous
import functools

import jax
import jax.numpy as jnp
from jax import lax
from jax.experimental import pallas as pl
from jax.experimental.pallas import tpu as pltpu
from jax.experimental.pallas import tpu_sc as plsc

N = 10000
E = 320000
D = 128

NC = 2
NS = 16
NW = NC * NS
E_PER_W = E // NW
CHUNK = 80
N_CHUNKS = E_PER_W // CHUNK
N_PAD = 10112
ROWS_PER_TILE = N_PAD // NS
ZCHUNK = 8

_LANES = 16
_VPR = D // _LANES


def _linear_body(x_ref, w_ref, b_ref, o_ref):
    o_ref[...] = (
        jnp.dot(x_ref[...], w_ref[...], preferred_element_type=jnp.float32)
        + b_ref[...]
    )


def _linear(x, W, b):
    bm = 2000
    return pl.pallas_call(
        _linear_body,
        grid=(N // bm,),
        in_specs=[
            pl.BlockSpec((bm, D), lambda i: (i, 0)),
            pl.BlockSpec((D, D), lambda i: (0, 0)),
            pl.BlockSpec((1, D), lambda i: (0, 0)),
        ],
        out_specs=pl.BlockSpec((bm, D), lambda i: (i, 0)),
        out_shape=jax.ShapeDtypeStruct((N, D), jnp.float32),
    )(x, W, b.reshape(1, D))


def _sc_spmm_body(h_hbm, src_hbm, dst_hbm, adj_hbm, out_hbm,
                  src_v, adj_v, dst_v, rows_v, acc_sh, sem):
    c = lax.axis_index("c")
    s = lax.axis_index("s")
    wid = s * NC + c

    zero16 = jnp.zeros((_LANES,), jnp.float32)
    for i in range(ZCHUNK):
        for j in range(_VPR):
            rows_v[i, pl.ds(j * _LANES, _LANES)] = zero16
    row0 = s * ROWS_PER_TILE

    def zero_body(i, _):
        pltpu.sync_copy(
            rows_v.at[pl.ds(0, ZCHUNK)],
            acc_sh.at[pl.ds(row0 + i * ZCHUNK, ZCHUNK)],
        )
        return ()

    lax.fori_loop(0, ROWS_PER_TILE // ZCHUNK, zero_body, ())
    plsc.subcore_barrier()

    e0 = wid * E_PER_W
    pltpu.sync_copy(src_hbm.at[pl.ds(e0, E_PER_W)], src_v)
    pltpu.sync_copy(adj_hbm.at[pl.ds(e0, E_PER_W)], adj_v)

    def chunk_body(t, _):
        base = t * CHUNK
        pltpu.sync_copy(dst_hbm.at[pl.ds(e0 + base, CHUNK)], dst_v)
        pltpu.async_copy(
            h_hbm.at[src_v.at[pl.ds(base, CHUNK)]], rows_v, sem
        ).wait()
        for g in range(CHUNK // _LANES):
            avec = adj_v[pl.ds(base + g * _LANES, _LANES)]
            for i in range(_LANES):
                a = jnp.full((_LANES,), avec[i], jnp.float32)
                r = g * _LANES + i
                for j in range(_VPR):
                    sl = pl.ds(j * _LANES, _LANES)
                    rows_v[r, sl] = rows_v[r, sl] * a
        pltpu.sync_copy(rows_v, acc_sh.at[dst_v], add=True)
        return ()

    lax.fori_loop(0, N_CHUNKS, chunk_body, ())
    plsc.subcore_barrier()

    pltpu.sync_copy(
        acc_sh.at[pl.ds(row0, ROWS_PER_TILE)],
        out_hbm.at[c].at[pl.ds(row0, ROWS_PER_TILE)],
    )


_sc_spmm = functools.partial(
    pl.kernel,
    out_type=jax.ShapeDtypeStruct((NC, N_PAD, D), jnp.float32),
    mesh=plsc.VectorSubcoreMesh(core_axis_name="c", subcore_axis_name="s"),
    scratch_types=[
        pltpu.VMEM((E_PER_W,), jnp.int32),
        pltpu.VMEM((E_PER_W,), jnp.float32),
        pltpu.VMEM((CHUNK,), jnp.int32),
        pltpu.VMEM((CHUNK, D), jnp.float32),
        pltpu.VMEM_SHARED((N_PAD, D), jnp.float32),
        pltpu.SemaphoreType.DMA,
    ],
)(_sc_spmm_body)


def _elu_body(p_ref, o_ref):
    t = p_ref[0] + p_ref[1]
    o_ref[...] = jnp.where(t > 0, t, jnp.exp(jnp.minimum(t, 0.0)) - 1.0)


def _elu_combine(partials):
    bm = 2000
    return pl.pallas_call(
        _elu_body,
        grid=(N // bm,),
        in_specs=[pl.BlockSpec((NC, bm, D), lambda i: (0, i, 0))],
        out_specs=pl.BlockSpec((bm, D), lambda i: (i, 0)),
        out_shape=jax.ShapeDtypeStruct((N, D), jnp.float32),
    )(partials)


def kernel(x, edge_index, adj_values, W, b):
    h = _linear(x, W, b)
    dst = edge_index[0]
    src = edge_index[1]
    partials = _sc_spmm(h, src, dst, adj_values)
    return _elu_combine(partials)


# speedup vs baseline: 7.8468x; 1.3576x over previous
import functools

import jax
import jax.numpy as jnp
from jax import lax
from jax.experimental import pallas as pl
from jax.experimental.pallas import tpu as pltpu
from jax.experimental.pallas import tpu_sc as plsc

N = 10000
E = 320000
D = 128

NC = 2
NS = 16
NW = NC * NS
E_PER_W = E // NW
CHUNK = 80
N_CHUNKS = E_PER_W // CHUNK
PAIR = 2 * CHUNK
N_PAD = 10112
ROWS_PER_TILE = N_PAD // NS
ZCHUNK = 8

_LANES = 16
_VPR = D // _LANES


def _linear_body(x_ref, w_ref, b_ref, o_ref):
    o_ref[...] = (
        jnp.dot(x_ref[...], w_ref[...], preferred_element_type=jnp.float32)
        + b_ref[...]
    )


def _linear(x, W, b):
    bm = 2000
    return pl.pallas_call(
        _linear_body,
        grid=(N // bm,),
        in_specs=[
            pl.BlockSpec((bm, D), lambda i: (i, 0)),
            pl.BlockSpec((D, D), lambda i: (0, 0)),
            pl.BlockSpec((1, D), lambda i: (0, 0)),
        ],
        out_specs=pl.BlockSpec((bm, D), lambda i: (i, 0)),
        out_shape=jax.ShapeDtypeStruct((N, D), jnp.float32),
    )(x, W, b.reshape(1, D))


def _sc_spmm_body(h_hbm, src_hbm, dst2_hbm, adj_hbm, out_hbm,
                  src_v, dst_v, adj0, adj1, rows0, rows1, acc_sh,
                  gsem0, gsem1, asem0, asem1):
    c = lax.axis_index("c")
    s = lax.axis_index("s")
    wid = s * NC + c

    zero16 = jnp.zeros((_LANES,), jnp.float32)
    for i in range(ZCHUNK):
        for j in range(_VPR):
            rows0[i, pl.ds(j * _LANES, _LANES)] = zero16
    row0 = s * ROWS_PER_TILE

    def zero_body(i, _):
        pltpu.sync_copy(
            rows0.at[pl.ds(0, ZCHUNK)],
            acc_sh.at[pl.ds(row0 + i * ZCHUNK, ZCHUNK)],
        )
        return ()

    lax.fori_loop(0, ROWS_PER_TILE // ZCHUNK, zero_body, ())

    e0 = wid * E_PER_W
    pltpu.sync_copy(src_hbm.at[pl.ds(e0, E_PER_W)], src_v)
    pltpu.sync_copy(dst2_hbm.at[wid], dst_v)
    plsc.subcore_barrier()

    def gather(t, rows, gsem):
        pltpu.async_copy(
            h_hbm.at[src_v.at[pl.ds(t * CHUNK, CHUNK)]], rows, gsem
        )

    def gather_wait(t, rows, gsem):
        pltpu.make_async_copy(
            h_hbm.at[src_v.at[pl.ds(t * CHUNK, CHUNK)]], rows, gsem
        ).wait()

    def adj_load(pair, abuf, asem):
        pltpu.async_copy(
            adj_hbm.at[pl.ds(e0 + pair * PAIR, PAIR)], abuf, asem
        )

    def adj_wait(abuf, asem):
        pltpu.make_async_copy(
            adj_hbm.at[pl.ds(e0, PAIR)], abuf, asem
        ).wait()

    adj_load(0, adj0, asem0)
    gather(0, rows0, gsem0)
    gather(1, rows1, gsem1)

    def chunk(t, rows, gsem, abuf, aoff):
        gather_wait(t, rows, gsem)
        for g in range(CHUNK // _LANES):
            avec = abuf[pl.ds(aoff + g * _LANES, _LANES)]
            for i in range(_LANES):
                a = jnp.full((_LANES,), avec[i], jnp.float32)
                r = g * _LANES + i
                for j in range(_VPR):
                    sl = pl.ds(j * _LANES, _LANES)
                    rows[r, sl] = rows[r, sl] * a

        pltpu.sync_copy(rows, acc_sh.at[dst_v.at[t]], add=True)

        @pl.when(t + 2 < N_CHUNKS)
        def _():
            gather(t + 2, rows, gsem)

    def quad_body(u, _):
        t0 = 4 * u
        adj_wait(adj0, asem0)
        adj_load(2 * u + 1, adj1, asem1)
        chunk(t0 + 0, rows0, gsem0, adj0, 0)
        chunk(t0 + 1, rows1, gsem1, adj0, CHUNK)
        adj_wait(adj1, asem1)

        @pl.when(u < (N_CHUNKS // 4) - 1)
        def _():
            adj_load(2 * u + 2, adj0, asem0)

        chunk(t0 + 2, rows0, gsem0, adj1, 0)
        chunk(t0 + 3, rows1, gsem1, adj1, CHUNK)
        return ()

    lax.fori_loop(0, N_CHUNKS // 4, quad_body, ())

    pltpu.sync_copy(
        adj_hbm.at[pl.ds(e0 + (N_CHUNKS - 1) * CHUNK, CHUNK)],
        adj0.at[pl.ds(0, CHUNK)],
    )
    chunk(N_CHUNKS - 1, rows0, gsem0, adj0, 0)

    plsc.subcore_barrier()
    pltpu.sync_copy(
        acc_sh.at[pl.ds(row0, ROWS_PER_TILE)],
        out_hbm.at[c].at[pl.ds(row0, ROWS_PER_TILE)],
    )


_sc_spmm = functools.partial(
    pl.kernel,
    out_type=jax.ShapeDtypeStruct((NC, N_PAD, D), jnp.float32),
    mesh=plsc.VectorSubcoreMesh(core_axis_name="c", subcore_axis_name="s"),
    scratch_types=[
        pltpu.VMEM((E_PER_W,), jnp.int32),
        pltpu.VMEM((N_CHUNKS, CHUNK), jnp.int32),
        pltpu.VMEM((PAIR,), jnp.float32),
        pltpu.VMEM((PAIR,), jnp.float32),
        pltpu.VMEM((CHUNK, D), jnp.float32),
        pltpu.VMEM((CHUNK, D), jnp.float32),
        pltpu.VMEM_SHARED((N_PAD, D), jnp.float32),
        pltpu.SemaphoreType.DMA,
        pltpu.SemaphoreType.DMA,
        pltpu.SemaphoreType.DMA,
        pltpu.SemaphoreType.DMA,
    ],
)(_sc_spmm_body)


def _elu_body(p_ref, o_ref):
    t = p_ref[0] + p_ref[1]
    o_ref[...] = jnp.where(t > 0, t, jnp.exp(jnp.minimum(t, 0.0)) - 1.0)


def _elu_combine(partials):
    bm = 2000
    return pl.pallas_call(
        _elu_body,
        grid=(N // bm,),
        in_specs=[pl.BlockSpec((NC, bm, D), lambda i: (0, i, 0))],
        out_specs=pl.BlockSpec((bm, D), lambda i: (i, 0)),
        out_shape=jax.ShapeDtypeStruct((N, D), jnp.float32),
    )(partials)


def kernel(x, edge_index, adj_values, W, b):
    h = _linear(x, W, b)
    dst = edge_index[0].reshape(NW, N_CHUNKS, CHUNK)
    src = edge_index[1]
    partials = _sc_spmm(h, src, dst, adj_values)
    return _elu_combine(partials)
